# baseline (speedup 1.0000x reference)
import jax
import jax.numpy as jnp
from jax.experimental import pallas as pl
from jax.experimental.pallas import tpu as pltpu
from jax.experimental.pallas import tpu_sc as plsc

_GATHER_WINDOW = 128
_REPACK_L = 18432


def _tc_repack(xt):
    n = xt.shape[1]
    l = _REPACK_L
    nblk = pl.cdiv(n // 2, l)

    def body(x_ref, o_ref):
        v = x_ref[...]
        o_ref[...] = jnp.concatenate([v[:, :l].T, v[:, l:].T], axis=-1)

    return pl.pallas_call(
        body,
        out_shape=jax.ShapeDtypeStruct((nblk * l, 128), xt.dtype),
        grid=(nblk,),
        in_specs=[pl.BlockSpec((64, 2 * l), lambda i: (0, i))],
        out_specs=pl.BlockSpec((l, 128), lambda i: (i, 0)),
        compiler_params=pltpu.CompilerParams(dimension_semantics=("parallel",)),
    )(xt)


def _sc_gather(table, idx2d):
    b = idx2d.shape[1]
    w = table.shape[1]
    mesh = plsc.VectorSubcoreMesh(core_axis_name="c", subcore_axis_name="s")

    @pl.kernel(
        out_type=jax.ShapeDtypeStruct((b, w), table.dtype),
        mesh=mesh,
    )
    def gather_kernel(x_hbm, i_hbm, o_hbm):
        def body(i_vmem, o_vmem):
            pltpu.sync_copy(x_hbm.at[i_vmem.at[0]], o_vmem)

        pltpu.emit_pipeline(
            body,
            grid=(b // _GATHER_WINDOW,),
            in_specs=[pl.BlockSpec((1, _GATHER_WINDOW), lambda i: (0, i))],
            out_specs=[pl.BlockSpec((_GATHER_WINDOW, w), lambda i: (i, 0))],
            core_axis_name=("c", "s"),
            dimension_semantics=(pltpu.PARALLEL,),
        )(i_hbm, o_hbm)

    return gather_kernel(table, idx2d)


def _tc_select_softmax(pairs, hi_col, a):
    blk = 2048

    def body(x_ref, s_ref, o_ref):
        v = x_ref[...]
        hi = s_ref[...] == 1
        sel = jnp.where(hi, v[:, a:], v[:, :a])
        m = jnp.max(sel, axis=-1, keepdims=True)
        e = jnp.exp(sel - m)
        o_ref[...] = e / jnp.sum(e, axis=-1, keepdims=True)

    n = pairs.shape[0]
    return pl.pallas_call(
        body,
        out_shape=jax.ShapeDtypeStruct((n, a), pairs.dtype),
        grid=(n // blk,),
        in_specs=[
            pl.BlockSpec((blk, pairs.shape[1]), lambda i: (i, 0)),
            pl.BlockSpec((blk, 1), lambda i: (i, 0)),
        ],
        out_specs=pl.BlockSpec((blk, a), lambda i: (i, 0)),
        compiler_params=pltpu.CompilerParams(dimension_semantics=("parallel",)),
    )(pairs, hi_col)


@jax.jit
def kernel(state, logits):
    n_rows, a = logits.shape
    l = _REPACK_L
    state = state.astype(jnp.int32)
    table = _tc_repack(logits.T)
    idx = (state // (2 * l)) * l + state % l
    hi = (state // l) & 1
    pairs = _sc_gather(table, idx.reshape(1, -1))
    return _tc_select_softmax(pairs, hi.reshape(-1, 1), a)


# speedup vs baseline: 1.0131x; 1.0131x over previous
import jax
import jax.numpy as jnp
from jax.experimental import pallas as pl
from jax.experimental.pallas import tpu as pltpu
from jax.experimental.pallas import tpu_sc as plsc

_GATHER_WINDOW = 128
_REPACK_L = 16384


def _tc_repack(xt):
    n = xt.shape[1]
    l = _REPACK_L
    nblk = pl.cdiv(n // 2, l)

    def body(x_ref, o_ref):
        v = x_ref[...]
        o_ref[...] = jnp.concatenate([v[:, :l].T, v[:, l:].T], axis=-1)

    return pl.pallas_call(
        body,
        out_shape=jax.ShapeDtypeStruct((nblk * l, 128), xt.dtype),
        grid=(nblk,),
        in_specs=[pl.BlockSpec((64, 2 * l), lambda i: (0, i))],
        out_specs=pl.BlockSpec((l, 128), lambda i: (i, 0)),
        compiler_params=pltpu.CompilerParams(dimension_semantics=("parallel",)),
    )(xt)


def _sc_gather(table, idx2d):
    b = idx2d.shape[1]
    w = table.shape[1]
    mesh = plsc.VectorSubcoreMesh(core_axis_name="c", subcore_axis_name="s")

    @pl.kernel(
        out_type=jax.ShapeDtypeStruct((b, w), table.dtype),
        mesh=mesh,
    )
    def gather_kernel(x_hbm, i_hbm, o_hbm):
        def body(i_vmem, o_vmem):
            pltpu.sync_copy(x_hbm.at[i_vmem.at[0]], o_vmem)

        pltpu.emit_pipeline(
            body,
            grid=(b // _GATHER_WINDOW,),
            in_specs=[pl.BlockSpec((1, _GATHER_WINDOW), lambda i: (0, i))],
            out_specs=[pl.BlockSpec((_GATHER_WINDOW, w), lambda i: (i, 0))],
            core_axis_name=("c", "s"),
            dimension_semantics=(pltpu.PARALLEL,),
        )(i_hbm, o_hbm)

    return gather_kernel(table, idx2d)


def _tc_select_softmax(pairs, hi_col, a):
    blk = 2048

    def body(x_ref, s_ref, o_ref):
        v = x_ref[...]
        hi = s_ref[...] == 1
        sel = jnp.where(hi, v[:, a:], v[:, :a])
        m = jnp.max(sel, axis=-1, keepdims=True)
        e = jnp.exp(sel - m)
        o_ref[...] = e / jnp.sum(e, axis=-1, keepdims=True)

    n = pairs.shape[0]
    return pl.pallas_call(
        body,
        out_shape=jax.ShapeDtypeStruct((n, a), pairs.dtype),
        grid=(n // blk,),
        in_specs=[
            pl.BlockSpec((blk, pairs.shape[1]), lambda i: (i, 0)),
            pl.BlockSpec((blk, 1), lambda i: (i, 0)),
        ],
        out_specs=pl.BlockSpec((blk, a), lambda i: (i, 0)),
        compiler_params=pltpu.CompilerParams(dimension_semantics=("parallel",)),
    )(pairs, hi_col)


@jax.jit
def kernel(state, logits):
    n_rows, a = logits.shape
    l = _REPACK_L
    state = state.astype(jnp.int32)
    table = _tc_repack(logits.T)
    idx = (state // (2 * l)) * l + state % l
    hi = (state // l) & 1
    pairs = _sc_gather(table, idx.reshape(1, -1))
    return _tc_select_softmax(pairs, hi.reshape(-1, 1), a)


# speedup vs baseline: 1.0549x; 1.0412x over previous
import jax
import jax.numpy as jnp
from jax.experimental import pallas as pl
from jax.experimental.pallas import tpu as pltpu
from jax.experimental.pallas import tpu_sc as plsc

_GATHER_WINDOW = 128
_REPACK_L = 16384


def _tc_repack(xt):
    n = xt.shape[1]
    l = _REPACK_L
    nblk = pl.cdiv(n // 2, l)

    def body(x_ref, o_ref):
        v = x_ref[...]
        o_ref[...] = jnp.concatenate([v[:, :l].T, v[:, l:].T], axis=-1)

    return pl.pallas_call(
        body,
        out_shape=jax.ShapeDtypeStruct((nblk * l, 128), xt.dtype),
        grid=(nblk,),
        in_specs=[pl.BlockSpec((64, 2 * l), lambda i: (0, i))],
        out_specs=pl.BlockSpec((l, 128), lambda i: (i, 0)),
        compiler_params=pltpu.CompilerParams(dimension_semantics=("parallel",)),
    )(xt)


def _sc_gather(table, idx2d):
    b = idx2d.shape[1]
    w = table.shape[1]
    mesh = plsc.VectorSubcoreMesh(core_axis_name="c", subcore_axis_name="s")

    @pl.kernel(
        out_type=jax.ShapeDtypeStruct((b, w), table.dtype),
        mesh=mesh,
    )
    def gather_kernel(x_hbm, i_hbm, o_hbm):
        def body(i_vmem, o_vmem):
            pltpu.sync_copy(x_hbm.at[i_vmem.at[0]], o_vmem)

        pltpu.emit_pipeline(
            body,
            grid=(b // _GATHER_WINDOW,),
            in_specs=[pl.BlockSpec((1, _GATHER_WINDOW), lambda i: (0, i))],
            out_specs=[pl.BlockSpec((_GATHER_WINDOW, w), lambda i: (i, 0))],
            core_axis_name=("c", "s"),
            dimension_semantics=(pltpu.PARALLEL,),
        )(i_hbm, o_hbm)

    return gather_kernel(table, idx2d)


def _tc_select_softmax(pairs, hi_col, a):
    blk = 2048

    def body(x_ref, h_ref, o_ref):
        v = x_ref[...]
        h = h_ref[...]
        lo = v[:, :a]
        sel = lo + h * (v[:, a:] - lo)
        st = sel.T
        m = jnp.max(st, axis=0, keepdims=True)
        e = jnp.exp(st - m)
        o_ref[...] = e / jnp.sum(e, axis=0, keepdims=True)

    n = pairs.shape[0]
    return pl.pallas_call(
        body,
        out_shape=jax.ShapeDtypeStruct((a, n), pairs.dtype),
        grid=(n // blk,),
        in_specs=[
            pl.BlockSpec((blk, pairs.shape[1]), lambda i: (i, 0)),
            pl.BlockSpec((blk, 1), lambda i: (i, 0)),
        ],
        out_specs=pl.BlockSpec((a, blk), lambda i: (0, i)),
        compiler_params=pltpu.CompilerParams(dimension_semantics=("arbitrary",)),
    )(pairs, hi_col)


@jax.jit
def kernel(state, logits):
    n_rows, a = logits.shape
    l = _REPACK_L
    state = state.astype(jnp.int32)
    table = _tc_repack(logits.T)
    idx = (state // (2 * l)) * l + state % l
    hi = ((state // l) & 1).astype(jnp.float32)
    pairs = _sc_gather(table, idx.reshape(1, -1))
    return _tc_select_softmax(pairs, hi.reshape(-1, 1), a).T


# speedup vs baseline: 1.0556x; 1.0006x over previous
import jax
import jax.numpy as jnp
from jax.experimental import pallas as pl
from jax.experimental.pallas import tpu as pltpu
from jax.experimental.pallas import tpu_sc as plsc

_GATHER_WINDOW = 128
_REPACK_L = 16384


def _tc_repack(xt):
    n = xt.shape[1]
    l = _REPACK_L
    nblk = pl.cdiv(n // 2, l)

    def body(x_ref, o_ref):
        o_ref[:, :64] = x_ref[:, :l][...].T
        o_ref[:, 64:] = x_ref[:, l:][...].T

    return pl.pallas_call(
        body,
        out_shape=jax.ShapeDtypeStruct((nblk * l, 128), xt.dtype),
        grid=(nblk,),
        in_specs=[pl.BlockSpec((64, 2 * l), lambda i: (0, i))],
        out_specs=pl.BlockSpec((l, 128), lambda i: (i, 0)),
        compiler_params=pltpu.CompilerParams(dimension_semantics=("parallel",)),
    )(xt)


def _sc_gather(table, idx2d):
    b = idx2d.shape[1]
    w = table.shape[1]
    mesh = plsc.VectorSubcoreMesh(core_axis_name="c", subcore_axis_name="s")

    @pl.kernel(
        out_type=jax.ShapeDtypeStruct((b, w), table.dtype),
        mesh=mesh,
    )
    def gather_kernel(x_hbm, i_hbm, o_hbm):
        def body(i_vmem, o_vmem):
            pltpu.sync_copy(x_hbm.at[i_vmem.at[0]], o_vmem)

        pltpu.emit_pipeline(
            body,
            grid=(b // _GATHER_WINDOW,),
            in_specs=[pl.BlockSpec((1, _GATHER_WINDOW), lambda i: (0, i))],
            out_specs=[pl.BlockSpec((_GATHER_WINDOW, w), lambda i: (i, 0))],
            core_axis_name=("c", "s"),
            dimension_semantics=(pltpu.PARALLEL,),
        )(i_hbm, o_hbm)

    return gather_kernel(table, idx2d)


def _tc_select_softmax(pairs, hi_col, a):
    blk = 2048

    def body(x_ref, h_ref, o_ref):
        v = x_ref[...]
        h = h_ref[...]
        lo = v[:, :a]
        sel = lo + h * (v[:, a:] - lo)
        st = sel.T
        m = jnp.max(st, axis=0, keepdims=True)
        e = jnp.exp(st - m)
        o_ref[...] = e / jnp.sum(e, axis=0, keepdims=True)

    n = pairs.shape[0]
    return pl.pallas_call(
        body,
        out_shape=jax.ShapeDtypeStruct((a, n), pairs.dtype),
        grid=(n // blk,),
        in_specs=[
            pl.BlockSpec((blk, pairs.shape[1]), lambda i: (i, 0)),
            pl.BlockSpec((blk, 1), lambda i: (i, 0)),
        ],
        out_specs=pl.BlockSpec((a, blk), lambda i: (0, i)),
        compiler_params=pltpu.CompilerParams(dimension_semantics=("arbitrary",)),
    )(pairs, hi_col)


@jax.jit
def kernel(state, logits):
    n_rows, a = logits.shape
    l = _REPACK_L
    state = state.astype(jnp.int32)
    table = _tc_repack(logits.T)
    idx = (state // (2 * l)) * l + state % l
    hi = ((state // l) & 1).astype(jnp.float32)
    pairs = _sc_gather(table, idx.reshape(1, -1))
    return _tc_select_softmax(pairs, hi.reshape(-1, 1), a).T


# speedup vs baseline: 1.0869x; 1.0297x over previous
import jax
import jax.numpy as jnp
from jax.experimental import pallas as pl
from jax.experimental.pallas import tpu as pltpu
from jax.experimental.pallas import tpu_sc as plsc

_GATHER_WINDOW = 128
_REPACK_L = 16384


def _tc_repack(xt):
    n = xt.shape[1]
    l = _REPACK_L
    nblk = pl.cdiv(n // 2, l)

    def body(x_ref, o_ref):
        o_ref[:, :64] = x_ref[:, :l][...].T
        o_ref[:, 64:] = x_ref[:, l:][...].T

    return pl.pallas_call(
        body,
        out_shape=jax.ShapeDtypeStruct((nblk * l, 128), xt.dtype),
        grid=(nblk,),
        in_specs=[pl.BlockSpec((64, 2 * l), lambda i: (0, i))],
        out_specs=pl.BlockSpec((l, 128), lambda i: (i, 0)),
        compiler_params=pltpu.CompilerParams(dimension_semantics=("parallel",)),
    )(xt)


def _sc_gather(table, idx2d):
    b = idx2d.shape[1]
    w = table.shape[1]
    mesh = plsc.VectorSubcoreMesh(core_axis_name="c", subcore_axis_name="s")

    lg = _REPACK_L.bit_length() - 1

    @pl.kernel(
        out_type=jax.ShapeDtypeStruct((b, w), table.dtype),
        mesh=mesh,
        scratch_types=[pltpu.VMEM((1, _GATHER_WINDOW), jnp.int32)],
    )
    def gather_kernel(x_hbm, i_hbm, o_hbm, idx_vmem):
        def body(i_vmem, o_vmem):
            @pl.loop(0, _GATHER_WINDOW, step=16)
            def _(k):
                s = i_vmem[0, pl.ds(k, 16)]
                idx_vmem[0, pl.ds(k, 16)] = ((s >> (lg + 1)) << lg) | (
                    s & (_REPACK_L - 1)
                )

            pltpu.sync_copy(x_hbm.at[idx_vmem.at[0]], o_vmem)

        pltpu.emit_pipeline(
            body,
            grid=(b // _GATHER_WINDOW,),
            in_specs=[pl.BlockSpec((1, _GATHER_WINDOW), lambda i: (0, i))],
            out_specs=[pl.BlockSpec((_GATHER_WINDOW, w), lambda i: (i, 0))],
            core_axis_name=("c", "s"),
            dimension_semantics=(pltpu.PARALLEL,),
        )(i_hbm, o_hbm)

    return gather_kernel(table, idx2d)


def _tc_select_softmax(pairs, state_mat, a):
    blk = 2048
    lg = _REPACK_L.bit_length() - 1

    def body(x_ref, s_ref, o_ref):
        vt = x_ref[...].T
        h = ((s_ref[0] >> lg) & 1).astype(pairs.dtype)
        lo = vt[:a]
        sel = lo + h * (vt[a:] - lo)
        m = jnp.max(sel, axis=0, keepdims=True)
        e = jnp.exp(sel - m)
        o_ref[...] = e / jnp.sum(e, axis=0, keepdims=True)

    n = pairs.shape[0]
    return pl.pallas_call(
        body,
        out_shape=jax.ShapeDtypeStruct((a, n), pairs.dtype),
        grid=(n // blk,),
        in_specs=[
            pl.BlockSpec((blk, pairs.shape[1]), lambda i: (i, 0)),
            pl.BlockSpec((1, 1, blk), lambda i: (i, 0, 0)),
        ],
        out_specs=pl.BlockSpec((a, blk), lambda i: (0, i)),
        compiler_params=pltpu.CompilerParams(dimension_semantics=("arbitrary",)),
    )(pairs, state_mat)


@jax.jit
def kernel(state, logits):
    n_rows, a = logits.shape
    state = state.astype(jnp.int32)
    table = _tc_repack(logits.T)
    pairs = _sc_gather(table, state.reshape(1, -1))
    return _tc_select_softmax(pairs, state.reshape(-1, 1, 2048), a).T
